# baseline (speedup 1.0000x reference)
import functools

import jax
import jax.numpy as jnp
from jax import lax
from jax.experimental import pallas as pl
from jax.experimental.pallas import tpu as pltpu
from jax.experimental.pallas import tpu_sc as plsc

N_NODES = 10000
N_PAD = 10240
N_EDGES = 160000
NC, NS = 2, 16
NW = NC * NS
EPW = N_EDGES // NW
CH = 128
NCHUNK = -(-EPW // CH)
EPWP = NCHUNK * CH
ZROWS = N_PAD // NS
TRASH = N_NODES + 8
EPS = 1e-5
BR = 1000
GRID = N_NODES // BR


def _sc_mesh():
    return plsc.VectorSubcoreMesh(
        core_axis_name="c", subcore_axis_name="s", num_cores=NC, num_subcores=NS
    )


@functools.partial(
    pl.kernel,
    out_type=jax.ShapeDtypeStruct((NC, N_PAD), jnp.float32),
    mesh=_sc_mesh(),
    scratch_types=[
        pltpu.VMEM((NCHUNK, CH), jnp.int32),
        pltpu.VMEM((CH,), jnp.float32),
        pltpu.VMEM_SHARED((N_PAD,), jnp.float32),
    ],
)
def _deg_kernel(dst_hbm, zeros_hbm, out_hbm, dstv, onesv, acc):
    c = lax.axis_index("c")
    s = lax.axis_index("s")
    wid = c * NS + s
    pltpu.sync_copy(zeros_hbm, acc.at[pl.ds(s * ZROWS, ZROWS)])
    pltpu.sync_copy(dst_hbm.at[wid], dstv)
    for i in range(CH // 16):
        onesv[pl.ds(i * 16, 16)] = jnp.ones((16,), jnp.float32)
    plsc.subcore_barrier()

    def body(j, carry):
        pltpu.sync_copy(onesv, acc.at[dstv.at[j]], add=True)
        return carry

    lax.fori_loop(0, NCHUNK, body, 0)
    plsc.subcore_barrier()
    pltpu.sync_copy(
        acc.at[pl.ds(s * ZROWS, ZROWS)], out_hbm.at[c, pl.ds(s * ZROWS, ZROWS)]
    )


NPAIR = NCHUNK // 2


def _make_scatter(d_feat, dtype=jnp.float32):
    @functools.partial(
        pl.kernel,
        out_type=jax.ShapeDtypeStruct((NC, N_PAD, d_feat), dtype),
        mesh=_sc_mesh(),
        compiler_params=pltpu.CompilerParams(use_tc_tiling_on_sc=False),
        scratch_types=[
            pltpu.VMEM((NCHUNK, CH), jnp.int32),
            pltpu.VMEM((NCHUNK, CH), jnp.int32),
            pltpu.VMEM((CH, d_feat), dtype),
            pltpu.VMEM((CH, d_feat), dtype),
            pltpu.VMEM_SHARED((N_PAD, d_feat), dtype),
            pltpu.SemaphoreType.DMA,
            pltpu.SemaphoreType.DMA,
        ],
    )
    def _scat(y_hbm, src_hbm, dst_hbm, zeros_hbm, out_hbm, srcv, dstv, rows0,
              rows1, acc, sem0, sem1):
        c = lax.axis_index("c")
        s = lax.axis_index("s")
        wid = c * NS + s
        pltpu.sync_copy(zeros_hbm, acc.at[pl.ds(s * ZROWS, ZROWS)])
        pltpu.sync_copy(src_hbm.at[wid], srcv)
        pltpu.sync_copy(dst_hbm.at[wid], dstv)
        plsc.subcore_barrier()
        pltpu.async_copy(y_hbm.at[srcv.at[0]], rows0, sem0)

        def body(p, carry):
            j0 = 2 * p
            pltpu.async_copy(y_hbm.at[srcv.at[j0 + 1]], rows1, sem1)
            pltpu.make_async_copy(y_hbm.at[srcv.at[j0]], rows0, sem0).wait()
            pltpu.sync_copy(rows0, acc.at[dstv.at[j0]], add=True)

            @pl.when(p < NPAIR - 1)
            def _():
                pltpu.async_copy(y_hbm.at[srcv.at[j0 + 2]], rows0, sem0)

            pltpu.make_async_copy(y_hbm.at[srcv.at[j0 + 1]], rows1, sem1).wait()
            pltpu.sync_copy(rows1, acc.at[dstv.at[j0 + 1]], add=True)
            return carry

        lax.fori_loop(0, NPAIR, body, 0)
        plsc.subcore_barrier()
        pltpu.sync_copy(
            acc.at[pl.ds(s * ZROWS, ZROWS)], out_hbm.at[c, pl.ds(s * ZROWS, ZROWS)]
        )

    return _scat


_scatter128 = _make_scatter(128, jnp.bfloat16)
_scatter64 = _make_scatter(64, jnp.bfloat16)


def _tc0_body(x_ref, w_ref, xw_ref):
    xw_ref[...] = jnp.dot(x_ref[...], w_ref[...], preferred_element_type=jnp.float32)


def _tc0(x, w1):
    return pl.pallas_call(
        _tc0_body,
        grid=(GRID,),
        in_specs=[
            pl.BlockSpec((BR, 256), lambda i: (i, 0)),
            pl.BlockSpec((256, 128), lambda i: (0, 0)),
        ],
        out_specs=pl.BlockSpec((BR, 128), lambda i: (i, 0)),
        out_shape=jax.ShapeDtypeStruct((N_NODES, 128), jnp.float32),
    )(x, w1)


def _tc1_body(xw_ref, da_ref, db_ref, y16_ref, dinv_ref):
    deg = da_ref[...] + db_ref[...] + 1.0
    dinv = 1.0 / jnp.sqrt(deg)
    y16_ref[...] = (xw_ref[...] * dinv).astype(jnp.bfloat16)
    dinv_ref[...] = dinv


def _tc1(xw, da, db):
    return pl.pallas_call(
        _tc1_body,
        grid=(GRID,),
        in_specs=[
            pl.BlockSpec((BR, 128), lambda i: (i, 0)),
            pl.BlockSpec((BR, 1), lambda i: (i, 0)),
            pl.BlockSpec((BR, 1), lambda i: (i, 0)),
        ],
        out_specs=[
            pl.BlockSpec((BR, 128), lambda i: (i, 0)),
            pl.BlockSpec((BR, 1), lambda i: (i, 0)),
        ],
        out_shape=[
            jax.ShapeDtypeStruct((N_NODES, 128), jnp.bfloat16),
            jax.ShapeDtypeStruct((N_NODES, 1), jnp.float32),
        ],
    )(xw, da, db)


def _bn_scale_shift(st_ref, g_ref, bt_ref):
    mean = st_ref[0:1, :] * (1.0 / N_NODES)
    var = st_ref[1:2, :] * (1.0 / N_NODES) - mean * mean
    scale = g_ref[...] / jnp.sqrt(var + EPS)
    shift = bt_ref[...] - mean * scale
    return scale, shift


def _pre_phase(acc_ref, yb_ref, dinv_ref, b_ref, pre_scr, st_scr):
    i = pl.program_id(1)
    a = acc_ref[0].astype(jnp.float32) + acc_ref[1].astype(jnp.float32)
    pre = (a + yb_ref[...].astype(jnp.float32)) * dinv_ref[...] + b_ref[...]
    pre_scr[pl.ds(i * BR, BR), :] = pre
    st = jnp.concatenate(
        [
            jnp.sum(pre, axis=0, keepdims=True),
            jnp.sum(pre * pre, axis=0, keepdims=True),
        ],
        axis=0,
    )

    @pl.when(i == 0)
    def _():
        st_scr[...] = st

    @pl.when(i > 0)
    def _():
        st_scr[...] += st


def _l1_body(acc_ref, yb_ref, dinv_ref, b_ref, g_ref, bt_ref, w_ref,
             h_ref, y2b_ref, pre_scr, st_scr):
    ph = pl.program_id(0)
    i = pl.program_id(1)

    @pl.when(ph == 0)
    def _():
        _pre_phase(acc_ref, yb_ref, dinv_ref, b_ref, pre_scr, st_scr)

    @pl.when(ph == 1)
    def _():
        scale, shift = _bn_scale_shift(st_scr, g_ref, bt_ref)
        h = jnp.maximum(pre_scr[pl.ds(i * BR, BR), :] * scale + shift, 0.0)
        h_ref[...] = h
        y2 = (
            jnp.dot(h, w_ref[...], preferred_element_type=jnp.float32)
            * dinv_ref[...]
        )
        y2b_ref[...] = y2.astype(jnp.bfloat16)


def _l1(acc, y1b, dinv, b, g, bt, w2):
    return pl.pallas_call(
        _l1_body,
        grid=(2, GRID),
        in_specs=[
            pl.BlockSpec((NC, BR, 128), lambda p, i: (0, i, 0)),
            pl.BlockSpec((BR, 128), lambda p, i: (i, 0)),
            pl.BlockSpec((BR, 1), lambda p, i: (i, 0)),
            pl.BlockSpec((1, 128), lambda p, i: (0, 0)),
            pl.BlockSpec((1, 128), lambda p, i: (0, 0)),
            pl.BlockSpec((1, 128), lambda p, i: (0, 0)),
            pl.BlockSpec((128, 64), lambda p, i: (0, 0)),
        ],
        out_specs=[
            pl.BlockSpec((BR, 128), lambda p, i: (i, 0)),
            pl.BlockSpec((BR, 64), lambda p, i: (i, 0)),
        ],
        out_shape=[
            jax.ShapeDtypeStruct((N_NODES, 128), jnp.float32),
            jax.ShapeDtypeStruct((N_NODES, 64), jnp.bfloat16),
        ],
        scratch_shapes=[
            pltpu.VMEM((N_NODES, 128), jnp.float32),
            pltpu.VMEM((2, 128), jnp.float32),
        ],
    )(acc, y1b, dinv, b, g, bt, w2)


def _l2_body(acc_ref, yb_ref, dinv_ref, b_ref, g_ref, bt_ref, x_ref, h1_ref,
             wf_ref, bf_ref, out_ref, pre_scr, st_scr):
    ph = pl.program_id(0)
    i = pl.program_id(1)

    @pl.when(ph == 0)
    def _():
        _pre_phase(acc_ref, yb_ref, dinv_ref, b_ref, pre_scr, st_scr)

    @pl.when(ph == 1)
    def _():
        scale, shift = _bn_scale_shift(st_scr, g_ref, bt_ref)
        h2 = jnp.maximum(pre_scr[pl.ds(i * BR, BR), :] * scale + shift, 0.0)
        o = (
            jnp.dot(x_ref[...], wf_ref[0:256, :], preferred_element_type=jnp.float32)
            + jnp.dot(h1_ref[...], wf_ref[256:384, :], preferred_element_type=jnp.float32)
            + jnp.dot(h2, wf_ref[384:448, :], preferred_element_type=jnp.float32)
            + bf_ref[...]
        )
        out_ref[...] = jnp.maximum(o, 0.0)


def _l2(acc, y2b, dinv, b, g, bt, x, h1, wf, bf):
    return pl.pallas_call(
        _l2_body,
        grid=(2, GRID),
        in_specs=[
            pl.BlockSpec((NC, BR, 64), lambda p, i: (0, i, 0)),
            pl.BlockSpec((BR, 64), lambda p, i: (i, 0)),
            pl.BlockSpec((BR, 1), lambda p, i: (i, 0)),
            pl.BlockSpec((1, 64), lambda p, i: (0, 0)),
            pl.BlockSpec((1, 64), lambda p, i: (0, 0)),
            pl.BlockSpec((1, 64), lambda p, i: (0, 0)),
            pl.BlockSpec((BR, 256), lambda p, i: (i, 0)),
            pl.BlockSpec((BR, 128), lambda p, i: (i, 0)),
            pl.BlockSpec((448, 256), lambda p, i: (0, 0)),
            pl.BlockSpec((1, 256), lambda p, i: (0, 0)),
        ],
        out_specs=pl.BlockSpec((BR, 256), lambda p, i: (i, 0)),
        out_shape=jax.ShapeDtypeStruct((N_NODES, 256), jnp.float32),
        scratch_shapes=[
            pltpu.VMEM((N_NODES, 64), jnp.float32),
            pltpu.VMEM((2, 64), jnp.float32),
        ],
    )(acc, y2b, dinv, b, g, bt, x, h1, wf, bf)


def kernel(node_features, edge_index, W1, b1, g1, bt1, W2, b2, g2, bt2, Wf, bf):
    x = node_features
    ei = edge_index.astype(jnp.int32)
    src3 = jnp.pad(ei[0].reshape(NW, EPW), ((0, 0), (0, EPWP - EPW))).reshape(
        NW, NCHUNK, CH
    )
    dst3 = jnp.pad(
        ei[1].reshape(NW, EPW), ((0, 0), (0, EPWP - EPW)), constant_values=TRASH
    ).reshape(NW, NCHUNK, CH)
    zd = jnp.zeros((ZROWS,), jnp.float32)
    z1 = jnp.zeros((ZROWS, 128), jnp.bfloat16)
    z2 = jnp.zeros((ZROWS, 64), jnp.bfloat16)

    degp = _deg_kernel(dst3, zd)
    xw1 = _tc0(x, W1)
    da = degp[0, :N_NODES].reshape(N_NODES, 1)
    db = degp[1, :N_NODES].reshape(N_NODES, 1)
    y1b, dinv = _tc1(xw1, da, db)
    acc1 = _scatter128(y1b, src3, dst3, z1)
    h1, y2b = _l1(
        acc1, y1b, dinv, b1.reshape(1, 128), g1.reshape(1, 128),
        bt1.reshape(1, 128), W2,
    )
    acc2 = _scatter64(y2b, src3, dst3, z2)
    return _l2(
        acc2, y2b, dinv, b2.reshape(1, 64), g2.reshape(1, 64),
        bt2.reshape(1, 64), x, h1, Wf, bf.reshape(1, 256),
    )


# speedup vs baseline: 1.0099x; 1.0099x over previous
import functools

import jax
import jax.numpy as jnp
from jax import lax
from jax.experimental import pallas as pl
from jax.experimental.pallas import tpu as pltpu
from jax.experimental.pallas import tpu_sc as plsc

N_NODES = 10000
N_PAD = 10240
N_EDGES = 160000
NC, NS = 2, 16
NW = NC * NS
EPW = N_EDGES // NW
CH = 128
NCHUNK = -(-EPW // CH)
EPWP = NCHUNK * CH
ZROWS = N_PAD // NS
TRASH = N_NODES + 8
EPS = 1e-5
BR = 1000
GRID = N_NODES // BR


def _sc_mesh():
    return plsc.VectorSubcoreMesh(
        core_axis_name="c", subcore_axis_name="s", num_cores=NC, num_subcores=NS
    )


@functools.partial(
    pl.kernel,
    out_type=jax.ShapeDtypeStruct((NC, N_PAD), jnp.float32),
    mesh=_sc_mesh(),
    scratch_types=[
        pltpu.VMEM((NCHUNK, CH), jnp.int32),
        pltpu.VMEM((CH,), jnp.float32),
        pltpu.VMEM_SHARED((N_PAD,), jnp.float32),
    ],
)
def _deg_kernel(dst_hbm, zeros_hbm, out_hbm, dstv, onesv, acc):
    c = lax.axis_index("c")
    s = lax.axis_index("s")
    wid = c * NS + s
    pltpu.sync_copy(zeros_hbm, acc.at[pl.ds(s * ZROWS, ZROWS)])
    pltpu.sync_copy(dst_hbm.at[wid], dstv)
    for i in range(CH // 16):
        onesv[pl.ds(i * 16, 16)] = jnp.ones((16,), jnp.float32)
    plsc.subcore_barrier()

    def body(j, carry):
        pltpu.sync_copy(onesv, acc.at[dstv.at[j]], add=True)
        return carry

    lax.fori_loop(0, NCHUNK, body, 0)
    plsc.subcore_barrier()
    pltpu.sync_copy(
        acc.at[pl.ds(s * ZROWS, ZROWS)], out_hbm.at[c, pl.ds(s * ZROWS, ZROWS)]
    )


NPAIR = NCHUNK // 2


def _make_scatter(d_feat, dtype=jnp.float32):
    @functools.partial(
        pl.kernel,
        out_type=jax.ShapeDtypeStruct((NC, N_PAD, d_feat), dtype),
        mesh=_sc_mesh(),
        compiler_params=pltpu.CompilerParams(use_tc_tiling_on_sc=False),
        scratch_types=[
            pltpu.VMEM((NCHUNK, CH), jnp.int32),
            pltpu.VMEM((NCHUNK, CH), jnp.int32),
            pltpu.VMEM((CH, d_feat), dtype),
            pltpu.VMEM((CH, d_feat), dtype),
            pltpu.VMEM_SHARED((N_PAD, d_feat), dtype),
            pltpu.SemaphoreType.DMA,
            pltpu.SemaphoreType.DMA,
        ],
    )
    def _scat(y_hbm, src_hbm, dst_hbm, zeros_hbm, out_hbm, srcv, dstv, rows0,
              rows1, acc, sem0, sem1):
        c = lax.axis_index("c")
        s = lax.axis_index("s")
        wid = c * NS + s
        pltpu.sync_copy(zeros_hbm, acc.at[pl.ds(s * ZROWS, ZROWS)])
        pltpu.sync_copy(src_hbm.at[wid], srcv)
        pltpu.sync_copy(dst_hbm.at[wid], dstv)
        plsc.subcore_barrier()
        pltpu.async_copy(y_hbm.at[srcv.at[0]], rows0, sem0)

        def body(p, carry):
            j0 = 2 * p
            pltpu.async_copy(y_hbm.at[srcv.at[j0 + 1]], rows1, sem1)
            pltpu.make_async_copy(y_hbm.at[srcv.at[j0]], rows0, sem0).wait()
            pltpu.sync_copy(rows0, acc.at[dstv.at[j0]], add=True)

            @pl.when(p < NPAIR - 1)
            def _():
                pltpu.async_copy(y_hbm.at[srcv.at[j0 + 2]], rows0, sem0)

            pltpu.make_async_copy(y_hbm.at[srcv.at[j0 + 1]], rows1, sem1).wait()
            pltpu.sync_copy(rows1, acc.at[dstv.at[j0 + 1]], add=True)
            return carry

        lax.fori_loop(0, NPAIR, body, 0)
        plsc.subcore_barrier()
        pltpu.sync_copy(
            acc.at[pl.ds(s * ZROWS, ZROWS)], out_hbm.at[c, pl.ds(s * ZROWS, ZROWS)]
        )

    return _scat


_scatter128 = _make_scatter(128, jnp.bfloat16)
_scatter64 = _make_scatter(64, jnp.bfloat16)


def _tc1_body(x_ref, w_ref, da_ref, db_ref, y16_ref, dinv_ref):
    deg = da_ref[...] + db_ref[...] + 1.0
    dinv = 1.0 / jnp.sqrt(deg)
    y = jnp.dot(x_ref[...], w_ref[...], preferred_element_type=jnp.float32) * dinv
    y16_ref[...] = y.astype(jnp.bfloat16)
    dinv_ref[...] = dinv


def _tc1(x, w1, da, db):
    return pl.pallas_call(
        _tc1_body,
        grid=(GRID,),
        in_specs=[
            pl.BlockSpec((BR, 256), lambda i: (i, 0)),
            pl.BlockSpec((256, 128), lambda i: (0, 0)),
            pl.BlockSpec((BR, 1), lambda i: (i, 0)),
            pl.BlockSpec((BR, 1), lambda i: (i, 0)),
        ],
        out_specs=[
            pl.BlockSpec((BR, 128), lambda i: (i, 0)),
            pl.BlockSpec((BR, 1), lambda i: (i, 0)),
        ],
        out_shape=[
            jax.ShapeDtypeStruct((N_NODES, 128), jnp.bfloat16),
            jax.ShapeDtypeStruct((N_NODES, 1), jnp.float32),
        ],
    )(x, w1, da, db)


def _bn_scale_shift(st_ref, g_ref, bt_ref):
    mean = st_ref[0:1, :] * (1.0 / N_NODES)
    var = st_ref[1:2, :] * (1.0 / N_NODES) - mean * mean
    scale = g_ref[...] / jnp.sqrt(var + EPS)
    shift = bt_ref[...] - mean * scale
    return scale, shift


def _pre_phase(acc_ref, yb_ref, dinv_ref, b_ref, pre_scr, st_scr):
    i = pl.program_id(1)
    a = acc_ref[0].astype(jnp.float32) + acc_ref[1].astype(jnp.float32)
    pre = (a + yb_ref[...].astype(jnp.float32)) * dinv_ref[...] + b_ref[...]
    pre_scr[pl.ds(i * BR, BR), :] = pre
    st = jnp.concatenate(
        [
            jnp.sum(pre, axis=0, keepdims=True),
            jnp.sum(pre * pre, axis=0, keepdims=True),
        ],
        axis=0,
    )

    @pl.when(i == 0)
    def _():
        st_scr[...] = st

    @pl.when(i > 0)
    def _():
        st_scr[...] += st


def _l1_body(acc_ref, yb_ref, dinv_ref, b_ref, g_ref, bt_ref, w_ref,
             h_ref, y2b_ref, pre_scr, st_scr):
    ph = pl.program_id(0)
    i = pl.program_id(1)

    @pl.when(ph == 0)
    def _():
        _pre_phase(acc_ref, yb_ref, dinv_ref, b_ref, pre_scr, st_scr)

    @pl.when(ph == 1)
    def _():
        scale, shift = _bn_scale_shift(st_scr, g_ref, bt_ref)
        h = jnp.maximum(pre_scr[pl.ds(i * BR, BR), :] * scale + shift, 0.0)
        h_ref[...] = h
        y2 = (
            jnp.dot(h, w_ref[...], preferred_element_type=jnp.float32)
            * dinv_ref[...]
        )
        y2b_ref[...] = y2.astype(jnp.bfloat16)


def _l1(acc, y1b, dinv, b, g, bt, w2):
    return pl.pallas_call(
        _l1_body,
        grid=(2, GRID),
        in_specs=[
            pl.BlockSpec((NC, BR, 128), lambda p, i: (0, i, 0)),
            pl.BlockSpec((BR, 128), lambda p, i: (i, 0)),
            pl.BlockSpec((BR, 1), lambda p, i: (i, 0)),
            pl.BlockSpec((1, 128), lambda p, i: (0, 0)),
            pl.BlockSpec((1, 128), lambda p, i: (0, 0)),
            pl.BlockSpec((1, 128), lambda p, i: (0, 0)),
            pl.BlockSpec((128, 64), lambda p, i: (0, 0)),
        ],
        out_specs=[
            pl.BlockSpec((BR, 128), lambda p, i: (i, 0)),
            pl.BlockSpec((BR, 64), lambda p, i: (i, 0)),
        ],
        out_shape=[
            jax.ShapeDtypeStruct((N_NODES, 128), jnp.float32),
            jax.ShapeDtypeStruct((N_NODES, 64), jnp.bfloat16),
        ],
        scratch_shapes=[
            pltpu.VMEM((N_NODES, 128), jnp.float32),
            pltpu.VMEM((2, 128), jnp.float32),
        ],
    )(acc, y1b, dinv, b, g, bt, w2)


def _l2_body(acc_ref, yb_ref, dinv_ref, b_ref, g_ref, bt_ref, x_ref, h1_ref,
             wf_ref, bf_ref, out_ref, pre_scr, st_scr):
    ph = pl.program_id(0)
    i = pl.program_id(1)

    @pl.when(ph == 0)
    def _():
        _pre_phase(acc_ref, yb_ref, dinv_ref, b_ref, pre_scr, st_scr)

    @pl.when(ph == 1)
    def _():
        scale, shift = _bn_scale_shift(st_scr, g_ref, bt_ref)
        h2 = jnp.maximum(pre_scr[pl.ds(i * BR, BR), :] * scale + shift, 0.0)
        o = (
            jnp.dot(x_ref[...], wf_ref[0:256, :], preferred_element_type=jnp.float32)
            + jnp.dot(h1_ref[...], wf_ref[256:384, :], preferred_element_type=jnp.float32)
            + jnp.dot(h2, wf_ref[384:448, :], preferred_element_type=jnp.float32)
            + bf_ref[...]
        )
        out_ref[...] = jnp.maximum(o, 0.0)


def _l2(acc, y2b, dinv, b, g, bt, x, h1, wf, bf):
    return pl.pallas_call(
        _l2_body,
        grid=(2, GRID),
        in_specs=[
            pl.BlockSpec((NC, BR, 64), lambda p, i: (0, i, 0)),
            pl.BlockSpec((BR, 64), lambda p, i: (i, 0)),
            pl.BlockSpec((BR, 1), lambda p, i: (i, 0)),
            pl.BlockSpec((1, 64), lambda p, i: (0, 0)),
            pl.BlockSpec((1, 64), lambda p, i: (0, 0)),
            pl.BlockSpec((1, 64), lambda p, i: (0, 0)),
            pl.BlockSpec((BR, 256), lambda p, i: (i, 0)),
            pl.BlockSpec((BR, 128), lambda p, i: (i, 0)),
            pl.BlockSpec((448, 256), lambda p, i: (0, 0)),
            pl.BlockSpec((1, 256), lambda p, i: (0, 0)),
        ],
        out_specs=pl.BlockSpec((BR, 256), lambda p, i: (i, 0)),
        out_shape=jax.ShapeDtypeStruct((N_NODES, 256), jnp.float32),
        scratch_shapes=[
            pltpu.VMEM((N_NODES, 64), jnp.float32),
            pltpu.VMEM((2, 64), jnp.float32),
        ],
    )(acc, y2b, dinv, b, g, bt, x, h1, wf, bf)


def kernel(node_features, edge_index, W1, b1, g1, bt1, W2, b2, g2, bt2, Wf, bf):
    x = node_features
    ei = edge_index.astype(jnp.int32)
    src3 = jnp.pad(ei[0].reshape(NW, EPW), ((0, 0), (0, EPWP - EPW))).reshape(
        NW, NCHUNK, CH
    )
    dst3 = jnp.pad(
        ei[1].reshape(NW, EPW), ((0, 0), (0, EPWP - EPW)), constant_values=TRASH
    ).reshape(NW, NCHUNK, CH)
    zd = jnp.zeros((ZROWS,), jnp.float32)
    z1 = jnp.zeros((ZROWS, 128), jnp.bfloat16)
    z2 = jnp.zeros((ZROWS, 64), jnp.bfloat16)

    degp = _deg_kernel(dst3, zd)
    da = degp[0, :N_NODES].reshape(N_NODES, 1)
    db = degp[1, :N_NODES].reshape(N_NODES, 1)
    y1b, dinv = _tc1(x, W1, da, db)
    acc1 = _scatter128(y1b, src3, dst3, z1)
    h1, y2b = _l1(
        acc1, y1b, dinv, b1.reshape(1, 128), g1.reshape(1, 128),
        bt1.reshape(1, 128), W2,
    )
    acc2 = _scatter64(y2b, src3, dst3, z2)
    return _l2(
        acc2, y2b, dinv, b2.reshape(1, 64), g2.reshape(1, 64),
        bt2.reshape(1, 64), x, h1, Wf, bf.reshape(1, 256),
    )


# speedup vs baseline: 1.0544x; 1.0441x over previous
import functools

import jax
import jax.numpy as jnp
from jax import lax
from jax.experimental import pallas as pl
from jax.experimental.pallas import tpu as pltpu
from jax.experimental.pallas import tpu_sc as plsc

N_NODES = 10000
N_PAD = 10240
N_EDGES = 160000
NC, NS = 2, 16
NW = NC * NS
EPW = N_EDGES // NW
CH = 128
NCHUNK = -(-EPW // CH)
EPWP = NCHUNK * CH
ZROWS = N_PAD // NS
TRASH = N_NODES + 8
EPS = 1e-5
BR = 1000
GRID = N_NODES // BR


def _sc_mesh():
    return plsc.VectorSubcoreMesh(
        core_axis_name="c", subcore_axis_name="s", num_cores=NC, num_subcores=NS
    )


@functools.partial(
    pl.kernel,
    out_type=jax.ShapeDtypeStruct((NC, N_PAD), jnp.float32),
    mesh=_sc_mesh(),
    scratch_types=[
        pltpu.VMEM((NCHUNK, CH), jnp.int32),
        pltpu.VMEM((CH,), jnp.float32),
        pltpu.VMEM_SHARED((N_PAD,), jnp.float32),
    ],
)
def _deg_kernel(dst_hbm, zeros_hbm, out_hbm, dstv, onesv, acc):
    c = lax.axis_index("c")
    s = lax.axis_index("s")
    wid = c * NS + s
    pltpu.sync_copy(zeros_hbm, acc.at[pl.ds(s * ZROWS, ZROWS)])
    pltpu.sync_copy(dst_hbm.at[wid], dstv)
    for i in range(CH // 16):
        onesv[pl.ds(i * 16, 16)] = jnp.ones((16,), jnp.float32)
    plsc.subcore_barrier()

    def body(j, carry):
        pltpu.sync_copy(onesv, acc.at[dstv.at[j]], add=True)
        return carry

    lax.fori_loop(0, NCHUNK, body, 0)
    plsc.subcore_barrier()
    pltpu.sync_copy(
        acc.at[pl.ds(s * ZROWS, ZROWS)], out_hbm.at[c, pl.ds(s * ZROWS, ZROWS)]
    )


NPAIR = NCHUNK // 2


def _make_scatter(d_feat, dtype=jnp.float32):
    @functools.partial(
        pl.kernel,
        out_type=jax.ShapeDtypeStruct((NC, N_PAD, d_feat), dtype),
        mesh=_sc_mesh(),
        compiler_params=pltpu.CompilerParams(use_tc_tiling_on_sc=False),
        scratch_types=[
            pltpu.VMEM((NCHUNK, CH), jnp.int32),
            pltpu.VMEM((NCHUNK, CH), jnp.int32),
            pltpu.VMEM((CH, d_feat), dtype),
            pltpu.VMEM((CH, d_feat), dtype),
            pltpu.VMEM((CH, d_feat), dtype),
            pltpu.VMEM((CH, d_feat), dtype),
            pltpu.VMEM_SHARED((N_PAD, d_feat), dtype),
            pltpu.SemaphoreType.DMA,
            pltpu.SemaphoreType.DMA,
            pltpu.SemaphoreType.DMA,
            pltpu.SemaphoreType.DMA,
        ],
    )
    def _scat(y_hbm, src_hbm, dst_hbm, zeros_hbm, out_hbm, srcv, dstv, r0, r1,
              r2, r3, acc, g0, g1, g2, g3):
        rows = (r0, r1, r2, r3)
        gsem = (g0, g1, g2, g3)
        c = lax.axis_index("c")
        s = lax.axis_index("s")
        wid = c * NS + s
        pltpu.sync_copy(zeros_hbm, acc.at[pl.ds(s * ZROWS, ZROWS)])
        pltpu.sync_copy(src_hbm.at[wid], srcv)
        pltpu.sync_copy(dst_hbm.at[wid], dstv)
        plsc.subcore_barrier()
        for b in range(4):
            pltpu.async_copy(y_hbm.at[srcv.at[b]], rows[b], gsem[b])

        def body(q, carry):
            for b in range(4):
                j = 4 * q + b
                pltpu.make_async_copy(y_hbm.at[srcv.at[j]], rows[b], gsem[b]).wait()
                pltpu.sync_copy(rows[b], acc.at[dstv.at[j]], add=True)

                @pl.when(j + 4 < NCHUNK)
                def _():
                    pltpu.async_copy(y_hbm.at[srcv.at[j + 4]], rows[b], gsem[b])

            return carry

        lax.fori_loop(0, NCHUNK // 4, body, 0)
        plsc.subcore_barrier()
        pltpu.sync_copy(
            acc.at[pl.ds(s * ZROWS, ZROWS)], out_hbm.at[c, pl.ds(s * ZROWS, ZROWS)]
        )

    return _scat


_scatter128 = _make_scatter(128, jnp.bfloat16)
_scatter64 = _make_scatter(64, jnp.bfloat16)


def _tc1_body(x_ref, w_ref, da_ref, db_ref, y16_ref, dinv_ref):
    deg = da_ref[...] + db_ref[...] + 1.0
    dinv = 1.0 / jnp.sqrt(deg)
    y = jnp.dot(x_ref[...], w_ref[...], preferred_element_type=jnp.float32) * dinv
    y16_ref[...] = y.astype(jnp.bfloat16)
    dinv_ref[...] = dinv


def _tc1(x, w1, da, db):
    return pl.pallas_call(
        _tc1_body,
        grid=(GRID,),
        in_specs=[
            pl.BlockSpec((BR, 256), lambda i: (i, 0)),
            pl.BlockSpec((256, 128), lambda i: (0, 0)),
            pl.BlockSpec((BR, 1), lambda i: (i, 0)),
            pl.BlockSpec((BR, 1), lambda i: (i, 0)),
        ],
        out_specs=[
            pl.BlockSpec((BR, 128), lambda i: (i, 0)),
            pl.BlockSpec((BR, 1), lambda i: (i, 0)),
        ],
        out_shape=[
            jax.ShapeDtypeStruct((N_NODES, 128), jnp.bfloat16),
            jax.ShapeDtypeStruct((N_NODES, 1), jnp.float32),
        ],
    )(x, w1, da, db)


def _bn_scale_shift(st_ref, g_ref, bt_ref):
    mean = st_ref[0:1, :] * (1.0 / N_NODES)
    var = st_ref[1:2, :] * (1.0 / N_NODES) - mean * mean
    scale = g_ref[...] / jnp.sqrt(var + EPS)
    shift = bt_ref[...] - mean * scale
    return scale, shift


def _pre_phase(acc_ref, yb_ref, dinv_ref, b_ref, pre_scr, st_scr):
    i = pl.program_id(1)
    a = acc_ref[0].astype(jnp.float32) + acc_ref[1].astype(jnp.float32)
    pre = (a + yb_ref[...].astype(jnp.float32)) * dinv_ref[...] + b_ref[...]
    pre_scr[pl.ds(i * BR, BR), :] = pre
    st = jnp.concatenate(
        [
            jnp.sum(pre, axis=0, keepdims=True),
            jnp.sum(pre * pre, axis=0, keepdims=True),
        ],
        axis=0,
    )

    @pl.when(i == 0)
    def _():
        st_scr[...] = st

    @pl.when(i > 0)
    def _():
        st_scr[...] += st


def _l1_body(acc_ref, yb_ref, dinv_ref, b_ref, g_ref, bt_ref, w_ref,
             h_ref, y2b_ref, pre_scr, st_scr):
    ph = pl.program_id(0)
    i = pl.program_id(1)

    @pl.when(ph == 0)
    def _():
        _pre_phase(acc_ref, yb_ref, dinv_ref, b_ref, pre_scr, st_scr)

    @pl.when(ph == 1)
    def _():
        scale, shift = _bn_scale_shift(st_scr, g_ref, bt_ref)
        h = jnp.maximum(pre_scr[pl.ds(i * BR, BR), :] * scale + shift, 0.0)
        h_ref[...] = h
        y2 = (
            jnp.dot(h, w_ref[...], preferred_element_type=jnp.float32)
            * dinv_ref[...]
        )
        y2b_ref[...] = y2.astype(jnp.bfloat16)


def _l1(acc, y1b, dinv, b, g, bt, w2):
    return pl.pallas_call(
        _l1_body,
        grid=(2, GRID),
        in_specs=[
            pl.BlockSpec((NC, BR, 128), lambda p, i: (0, i, 0)),
            pl.BlockSpec((BR, 128), lambda p, i: (i, 0)),
            pl.BlockSpec((BR, 1), lambda p, i: (i, 0)),
            pl.BlockSpec((1, 128), lambda p, i: (0, 0)),
            pl.BlockSpec((1, 128), lambda p, i: (0, 0)),
            pl.BlockSpec((1, 128), lambda p, i: (0, 0)),
            pl.BlockSpec((128, 64), lambda p, i: (0, 0)),
        ],
        out_specs=[
            pl.BlockSpec((BR, 128), lambda p, i: (i, 0)),
            pl.BlockSpec((BR, 64), lambda p, i: (i, 0)),
        ],
        out_shape=[
            jax.ShapeDtypeStruct((N_NODES, 128), jnp.float32),
            jax.ShapeDtypeStruct((N_NODES, 64), jnp.bfloat16),
        ],
        scratch_shapes=[
            pltpu.VMEM((N_NODES, 128), jnp.float32),
            pltpu.VMEM((2, 128), jnp.float32),
        ],
    )(acc, y1b, dinv, b, g, bt, w2)


def _l2_body(acc_ref, yb_ref, dinv_ref, b_ref, g_ref, bt_ref, x_ref, h1_ref,
             wf_ref, bf_ref, out_ref, pre_scr, st_scr):
    ph = pl.program_id(0)
    i = pl.program_id(1)

    @pl.when(ph == 0)
    def _():
        _pre_phase(acc_ref, yb_ref, dinv_ref, b_ref, pre_scr, st_scr)

    @pl.when(ph == 1)
    def _():
        scale, shift = _bn_scale_shift(st_scr, g_ref, bt_ref)
        h2 = jnp.maximum(pre_scr[pl.ds(i * BR, BR), :] * scale + shift, 0.0)
        o = (
            jnp.dot(x_ref[...], wf_ref[0:256, :], preferred_element_type=jnp.float32)
            + jnp.dot(h1_ref[...], wf_ref[256:384, :], preferred_element_type=jnp.float32)
            + jnp.dot(h2, wf_ref[384:448, :], preferred_element_type=jnp.float32)
            + bf_ref[...]
        )
        out_ref[...] = jnp.maximum(o, 0.0)


def _l2(acc, y2b, dinv, b, g, bt, x, h1, wf, bf):
    return pl.pallas_call(
        _l2_body,
        grid=(2, GRID),
        in_specs=[
            pl.BlockSpec((NC, BR, 64), lambda p, i: (0, i, 0)),
            pl.BlockSpec((BR, 64), lambda p, i: (i, 0)),
            pl.BlockSpec((BR, 1), lambda p, i: (i, 0)),
            pl.BlockSpec((1, 64), lambda p, i: (0, 0)),
            pl.BlockSpec((1, 64), lambda p, i: (0, 0)),
            pl.BlockSpec((1, 64), lambda p, i: (0, 0)),
            pl.BlockSpec((BR, 256), lambda p, i: (i, 0)),
            pl.BlockSpec((BR, 128), lambda p, i: (i, 0)),
            pl.BlockSpec((448, 256), lambda p, i: (0, 0)),
            pl.BlockSpec((1, 256), lambda p, i: (0, 0)),
        ],
        out_specs=pl.BlockSpec((BR, 256), lambda p, i: (i, 0)),
        out_shape=jax.ShapeDtypeStruct((N_NODES, 256), jnp.float32),
        scratch_shapes=[
            pltpu.VMEM((N_NODES, 64), jnp.float32),
            pltpu.VMEM((2, 64), jnp.float32),
        ],
    )(acc, y2b, dinv, b, g, bt, x, h1, wf, bf)


def kernel(node_features, edge_index, W1, b1, g1, bt1, W2, b2, g2, bt2, Wf, bf):
    x = node_features
    ei = edge_index.astype(jnp.int32)
    src3 = jnp.pad(ei[0].reshape(NW, EPW), ((0, 0), (0, EPWP - EPW))).reshape(
        NW, NCHUNK, CH
    )
    dst3 = jnp.pad(
        ei[1].reshape(NW, EPW), ((0, 0), (0, EPWP - EPW)), constant_values=TRASH
    ).reshape(NW, NCHUNK, CH)
    zd = jnp.zeros((ZROWS,), jnp.float32)
    z1 = jnp.zeros((ZROWS, 128), jnp.bfloat16)
    z2 = jnp.zeros((ZROWS, 64), jnp.bfloat16)

    degp = _deg_kernel(dst3, zd)
    da = degp[0, :N_NODES].reshape(N_NODES, 1)
    db = degp[1, :N_NODES].reshape(N_NODES, 1)
    y1b, dinv = _tc1(x, W1, da, db)
    acc1 = _scatter128(y1b, src3, dst3, z1)
    h1, y2b = _l1(
        acc1, y1b, dinv, b1.reshape(1, 128), g1.reshape(1, 128),
        bt1.reshape(1, 128), W2,
    )
    acc2 = _scatter64(y2b, src3, dst3, z2)
    return _l2(
        acc2, y2b, dinv, b2.reshape(1, 64), g2.reshape(1, 64),
        bt2.reshape(1, 64), x, h1, Wf, bf.reshape(1, 256),
    )


# speedup vs baseline: 1.0610x; 1.0063x over previous
import functools

import jax
import jax.numpy as jnp
from jax import lax
from jax.experimental import pallas as pl
from jax.experimental.pallas import tpu as pltpu
from jax.experimental.pallas import tpu_sc as plsc

N_NODES = 10000
N_PAD = 10240
N_EDGES = 160000
NC, NS = 2, 16
NW = NC * NS
EPW = N_EDGES // NW
CH = 128
NCHUNK = -(-EPW // CH)
EPWP = NCHUNK * CH
ZROWS = N_PAD // NS
TRASH = N_NODES + 8
EPS = 1e-5
BR = 1000
GRID = N_NODES // BR


def _sc_mesh():
    return plsc.VectorSubcoreMesh(
        core_axis_name="c", subcore_axis_name="s", num_cores=NC, num_subcores=NS
    )


@functools.partial(
    pl.kernel,
    out_type=jax.ShapeDtypeStruct((NC, N_PAD), jnp.float32),
    mesh=_sc_mesh(),
    scratch_types=[
        pltpu.VMEM((NCHUNK, CH), jnp.int32),
        pltpu.VMEM((CH,), jnp.float32),
        pltpu.VMEM_SHARED((N_PAD,), jnp.float32),
    ],
)
def _deg_kernel(dst_hbm, zeros_hbm, out_hbm, dstv, onesv, acc):
    c = lax.axis_index("c")
    s = lax.axis_index("s")
    wid = c * NS + s
    pltpu.sync_copy(zeros_hbm, acc.at[pl.ds(s * ZROWS, ZROWS)])
    pltpu.sync_copy(dst_hbm.at[wid], dstv)
    for i in range(CH // 16):
        onesv[pl.ds(i * 16, 16)] = jnp.ones((16,), jnp.float32)
    plsc.subcore_barrier()

    def body(j, carry):
        pltpu.sync_copy(onesv, acc.at[dstv.at[j]], add=True)
        return carry

    lax.fori_loop(0, NCHUNK, body, 0)
    plsc.subcore_barrier()
    pltpu.sync_copy(
        acc.at[pl.ds(s * ZROWS, ZROWS)], out_hbm.at[c, pl.ds(s * ZROWS, ZROWS)]
    )


NPAIR = NCHUNK // 2


def _make_scatter(d_feat, dtype=jnp.float32, nbuf=4):
    @functools.partial(
        pl.kernel,
        out_type=jax.ShapeDtypeStruct((NC, N_PAD, d_feat), dtype),
        mesh=_sc_mesh(),
        compiler_params=pltpu.CompilerParams(use_tc_tiling_on_sc=False),
        scratch_types=[
            pltpu.VMEM((NCHUNK, CH), jnp.int32),
            pltpu.VMEM((NCHUNK, CH), jnp.int32),
        ]
        + [pltpu.VMEM((CH, d_feat), dtype) for _ in range(nbuf)]
        + [pltpu.VMEM_SHARED((N_PAD, d_feat), dtype)]
        + [pltpu.SemaphoreType.DMA for _ in range(nbuf)],
    )
    def _scat(y_hbm, src_hbm, dst_hbm, zeros_hbm, out_hbm, srcv, dstv, *rest):
        rows = rest[:nbuf]
        acc = rest[nbuf]
        gsem = rest[nbuf + 1:]
        c = lax.axis_index("c")
        s = lax.axis_index("s")
        wid = c * NS + s
        pltpu.sync_copy(zeros_hbm, acc.at[pl.ds(s * ZROWS, ZROWS)])
        pltpu.sync_copy(src_hbm.at[wid], srcv)
        pltpu.sync_copy(dst_hbm.at[wid], dstv)
        plsc.subcore_barrier()
        for b in range(nbuf):
            pltpu.async_copy(y_hbm.at[srcv.at[b]], rows[b], gsem[b])

        def body(q, carry):
            for b in range(nbuf):
                j = nbuf * q + b
                pltpu.make_async_copy(y_hbm.at[srcv.at[j]], rows[b], gsem[b]).wait()
                pltpu.sync_copy(rows[b], acc.at[dstv.at[j]], add=True)

                @pl.when(j + nbuf < NCHUNK)
                def _():
                    pltpu.async_copy(y_hbm.at[srcv.at[j + nbuf]], rows[b], gsem[b])

            return carry

        lax.fori_loop(0, NCHUNK // nbuf, body, 0)
        for j in range(NCHUNK - NCHUNK % nbuf, NCHUNK):
            b = j % nbuf
            pltpu.make_async_copy(y_hbm.at[srcv.at[j]], rows[b], gsem[b]).wait()
            pltpu.sync_copy(rows[b], acc.at[dstv.at[j]], add=True)
        plsc.subcore_barrier()
        pltpu.sync_copy(
            acc.at[pl.ds(s * ZROWS, ZROWS)], out_hbm.at[c, pl.ds(s * ZROWS, ZROWS)]
        )

    return _scat


_scatter128 = _make_scatter(128, jnp.bfloat16, nbuf=4)
_scatter64 = _make_scatter(64, jnp.bfloat16, nbuf=8)


def _tc1_body(x_ref, w_ref, da_ref, db_ref, y16_ref, dinv_ref):
    deg = da_ref[...] + db_ref[...] + 1.0
    dinv = 1.0 / jnp.sqrt(deg)
    y = jnp.dot(x_ref[...], w_ref[...], preferred_element_type=jnp.float32) * dinv
    y16_ref[...] = y.astype(jnp.bfloat16)
    dinv_ref[...] = dinv


def _tc1(x, w1, da, db):
    return pl.pallas_call(
        _tc1_body,
        grid=(GRID,),
        in_specs=[
            pl.BlockSpec((BR, 256), lambda i: (i, 0)),
            pl.BlockSpec((256, 128), lambda i: (0, 0)),
            pl.BlockSpec((BR, 1), lambda i: (i, 0)),
            pl.BlockSpec((BR, 1), lambda i: (i, 0)),
        ],
        out_specs=[
            pl.BlockSpec((BR, 128), lambda i: (i, 0)),
            pl.BlockSpec((BR, 1), lambda i: (i, 0)),
        ],
        out_shape=[
            jax.ShapeDtypeStruct((N_NODES, 128), jnp.bfloat16),
            jax.ShapeDtypeStruct((N_NODES, 1), jnp.float32),
        ],
    )(x, w1, da, db)


def _bn_scale_shift(st_ref, g_ref, bt_ref):
    mean = st_ref[0:1, :] * (1.0 / N_NODES)
    var = st_ref[1:2, :] * (1.0 / N_NODES) - mean * mean
    scale = g_ref[...] / jnp.sqrt(var + EPS)
    shift = bt_ref[...] - mean * scale
    return scale, shift


def _pre_phase(acc_ref, yb_ref, dinv_ref, b_ref, pre_scr, st_scr):
    i = pl.program_id(1)
    a = acc_ref[0].astype(jnp.float32) + acc_ref[1].astype(jnp.float32)
    pre = (a + yb_ref[...].astype(jnp.float32)) * dinv_ref[...] + b_ref[...]
    pre_scr[pl.ds(i * BR, BR), :] = pre
    st = jnp.concatenate(
        [
            jnp.sum(pre, axis=0, keepdims=True),
            jnp.sum(pre * pre, axis=0, keepdims=True),
        ],
        axis=0,
    )

    @pl.when(i == 0)
    def _():
        st_scr[...] = st

    @pl.when(i > 0)
    def _():
        st_scr[...] += st


def _l1_body(acc_ref, yb_ref, dinv_ref, b_ref, g_ref, bt_ref, w_ref,
             h_ref, y2b_ref, pre_scr, st_scr):
    ph = pl.program_id(0)
    i = pl.program_id(1)

    @pl.when(ph == 0)
    def _():
        _pre_phase(acc_ref, yb_ref, dinv_ref, b_ref, pre_scr, st_scr)

    @pl.when(ph == 1)
    def _():
        scale, shift = _bn_scale_shift(st_scr, g_ref, bt_ref)
        h = jnp.maximum(pre_scr[pl.ds(i * BR, BR), :] * scale + shift, 0.0)
        h_ref[...] = h
        y2 = (
            jnp.dot(h, w_ref[...], preferred_element_type=jnp.float32)
            * dinv_ref[...]
        )
        y2b_ref[...] = y2.astype(jnp.bfloat16)


def _l1(acc, y1b, dinv, b, g, bt, w2):
    return pl.pallas_call(
        _l1_body,
        grid=(2, GRID),
        in_specs=[
            pl.BlockSpec((NC, BR, 128), lambda p, i: (0, i, 0)),
            pl.BlockSpec((BR, 128), lambda p, i: (i, 0)),
            pl.BlockSpec((BR, 1), lambda p, i: (i, 0)),
            pl.BlockSpec((1, 128), lambda p, i: (0, 0)),
            pl.BlockSpec((1, 128), lambda p, i: (0, 0)),
            pl.BlockSpec((1, 128), lambda p, i: (0, 0)),
            pl.BlockSpec((128, 64), lambda p, i: (0, 0)),
        ],
        out_specs=[
            pl.BlockSpec((BR, 128), lambda p, i: (i, 0)),
            pl.BlockSpec((BR, 64), lambda p, i: (i, 0)),
        ],
        out_shape=[
            jax.ShapeDtypeStruct((N_NODES, 128), jnp.float32),
            jax.ShapeDtypeStruct((N_NODES, 64), jnp.bfloat16),
        ],
        scratch_shapes=[
            pltpu.VMEM((N_NODES, 128), jnp.float32),
            pltpu.VMEM((2, 128), jnp.float32),
        ],
    )(acc, y1b, dinv, b, g, bt, w2)


def _l2_body(acc_ref, yb_ref, dinv_ref, b_ref, g_ref, bt_ref, x_ref, h1_ref,
             wf_ref, bf_ref, out_ref, pre_scr, st_scr):
    ph = pl.program_id(0)
    i = pl.program_id(1)

    @pl.when(ph == 0)
    def _():
        _pre_phase(acc_ref, yb_ref, dinv_ref, b_ref, pre_scr, st_scr)

    @pl.when(ph == 1)
    def _():
        scale, shift = _bn_scale_shift(st_scr, g_ref, bt_ref)
        h2 = jnp.maximum(pre_scr[pl.ds(i * BR, BR), :] * scale + shift, 0.0)
        o = (
            jnp.dot(x_ref[...], wf_ref[0:256, :], preferred_element_type=jnp.float32)
            + jnp.dot(h1_ref[...], wf_ref[256:384, :], preferred_element_type=jnp.float32)
            + jnp.dot(h2, wf_ref[384:448, :], preferred_element_type=jnp.float32)
            + bf_ref[...]
        )
        out_ref[...] = jnp.maximum(o, 0.0)


def _l2(acc, y2b, dinv, b, g, bt, x, h1, wf, bf):
    return pl.pallas_call(
        _l2_body,
        grid=(2, GRID),
        in_specs=[
            pl.BlockSpec((NC, BR, 64), lambda p, i: (0, i, 0)),
            pl.BlockSpec((BR, 64), lambda p, i: (i, 0)),
            pl.BlockSpec((BR, 1), lambda p, i: (i, 0)),
            pl.BlockSpec((1, 64), lambda p, i: (0, 0)),
            pl.BlockSpec((1, 64), lambda p, i: (0, 0)),
            pl.BlockSpec((1, 64), lambda p, i: (0, 0)),
            pl.BlockSpec((BR, 256), lambda p, i: (i, 0)),
            pl.BlockSpec((BR, 128), lambda p, i: (i, 0)),
            pl.BlockSpec((448, 256), lambda p, i: (0, 0)),
            pl.BlockSpec((1, 256), lambda p, i: (0, 0)),
        ],
        out_specs=pl.BlockSpec((BR, 256), lambda p, i: (i, 0)),
        out_shape=jax.ShapeDtypeStruct((N_NODES, 256), jnp.float32),
        scratch_shapes=[
            pltpu.VMEM((N_NODES, 64), jnp.float32),
            pltpu.VMEM((2, 64), jnp.float32),
        ],
    )(acc, y2b, dinv, b, g, bt, x, h1, wf, bf)


def kernel(node_features, edge_index, W1, b1, g1, bt1, W2, b2, g2, bt2, Wf, bf):
    x = node_features
    ei = edge_index.astype(jnp.int32)
    src3 = jnp.pad(ei[0].reshape(NW, EPW), ((0, 0), (0, EPWP - EPW))).reshape(
        NW, NCHUNK, CH
    )
    dst3 = jnp.pad(
        ei[1].reshape(NW, EPW), ((0, 0), (0, EPWP - EPW)), constant_values=TRASH
    ).reshape(NW, NCHUNK, CH)
    zd = jnp.zeros((ZROWS,), jnp.float32)
    z1 = jnp.zeros((ZROWS, 128), jnp.bfloat16)
    z2 = jnp.zeros((ZROWS, 64), jnp.bfloat16)

    degp = _deg_kernel(dst3, zd)
    da = degp[0, :N_NODES].reshape(N_NODES, 1)
    db = degp[1, :N_NODES].reshape(N_NODES, 1)
    y1b, dinv = _tc1(x, W1, da, db)
    acc1 = _scatter128(y1b, src3, dst3, z1)
    h1, y2b = _l1(
        acc1, y1b, dinv, b1.reshape(1, 128), g1.reshape(1, 128),
        bt1.reshape(1, 128), W2,
    )
    acc2 = _scatter64(y2b, src3, dst3, z2)
    return _l2(
        acc2, y2b, dinv, b2.reshape(1, 64), g2.reshape(1, 64),
        bt2.reshape(1, 64), x, h1, Wf, bf.reshape(1, 256),
    )


# speedup vs baseline: 1.0621x; 1.0011x over previous
import functools

import jax
import jax.numpy as jnp
from jax import lax
from jax.experimental import pallas as pl
from jax.experimental.pallas import tpu as pltpu
from jax.experimental.pallas import tpu_sc as plsc

N_NODES = 10000
N_PAD = 10240
N_EDGES = 160000
NC, NS = 2, 16
NW = NC * NS
EPW = N_EDGES // NW
CH = 128
NCHUNK = -(-EPW // CH)
EPWP = NCHUNK * CH
ZROWS = N_PAD // NS
TRASH = N_NODES + 8
EPS = 1e-5
BR = 1000
GRID = N_NODES // BR


def _sc_mesh():
    return plsc.VectorSubcoreMesh(
        core_axis_name="c", subcore_axis_name="s", num_cores=NC, num_subcores=NS
    )


@functools.partial(
    pl.kernel,
    out_type=jax.ShapeDtypeStruct((NC, N_PAD), jnp.float32),
    mesh=_sc_mesh(),
    scratch_types=[
        pltpu.VMEM((NCHUNK, CH), jnp.int32),
        pltpu.VMEM((CH,), jnp.float32),
        pltpu.VMEM_SHARED((N_PAD,), jnp.float32),
    ],
)
def _deg_kernel(dst_hbm, zeros_hbm, out_hbm, dstv, onesv, acc):
    c = lax.axis_index("c")
    s = lax.axis_index("s")
    wid = c * NS + s
    pltpu.sync_copy(zeros_hbm, acc.at[pl.ds(s * ZROWS, ZROWS)])
    pltpu.sync_copy(dst_hbm.at[wid], dstv)
    for i in range(CH // 16):
        onesv[pl.ds(i * 16, 16)] = jnp.ones((16,), jnp.float32)
    plsc.subcore_barrier()

    def body(j, carry):
        pltpu.sync_copy(onesv, acc.at[dstv.at[j]], add=True)
        return carry

    lax.fori_loop(0, NCHUNK, body, 0)
    plsc.subcore_barrier()
    pltpu.sync_copy(
        acc.at[pl.ds(s * ZROWS, ZROWS)], out_hbm.at[c, pl.ds(s * ZROWS, ZROWS)]
    )


NPAIR = NCHUNK // 2


def _make_scatter(d_feat, dtype=jnp.float32, nbuf=4):
    @functools.partial(
        pl.kernel,
        out_type=jax.ShapeDtypeStruct((NC, N_PAD, d_feat), dtype),
        mesh=_sc_mesh(),
        compiler_params=pltpu.CompilerParams(use_tc_tiling_on_sc=False),
        scratch_types=[
            pltpu.VMEM((NCHUNK, CH), jnp.int32),
            pltpu.VMEM((NCHUNK, CH), jnp.int32),
        ]
        + [pltpu.VMEM((CH, d_feat), dtype) for _ in range(nbuf)]
        + [pltpu.VMEM_SHARED((N_PAD, d_feat), dtype)]
        + [pltpu.SemaphoreType.DMA for _ in range(nbuf)],
    )
    def _scat(y_hbm, src_hbm, dst_hbm, zeros_hbm, out_hbm, srcv, dstv, *rest):
        rows = rest[:nbuf]
        acc = rest[nbuf]
        gsem = rest[nbuf + 1:]
        c = lax.axis_index("c")
        s = lax.axis_index("s")
        wid = c * NS + s
        pltpu.sync_copy(zeros_hbm, acc.at[pl.ds(s * ZROWS, ZROWS)])
        pltpu.sync_copy(src_hbm.at[wid], srcv)
        pltpu.sync_copy(dst_hbm.at[wid], dstv)
        plsc.subcore_barrier()
        for b in range(nbuf):
            pltpu.async_copy(y_hbm.at[srcv.at[b]], rows[b], gsem[b])

        def body(q, carry):
            for b in range(nbuf):
                j = nbuf * q + b
                pltpu.make_async_copy(y_hbm.at[srcv.at[j]], rows[b], gsem[b]).wait()
                pltpu.sync_copy(rows[b], acc.at[dstv.at[j]], add=True)

                @pl.when(j + nbuf < NCHUNK)
                def _():
                    pltpu.async_copy(y_hbm.at[srcv.at[j + nbuf]], rows[b], gsem[b])

            return carry

        lax.fori_loop(0, NCHUNK // nbuf, body, 0)
        for j in range(NCHUNK - NCHUNK % nbuf, NCHUNK):
            b = j % nbuf
            pltpu.make_async_copy(y_hbm.at[srcv.at[j]], rows[b], gsem[b]).wait()
            pltpu.sync_copy(rows[b], acc.at[dstv.at[j]], add=True)
        plsc.subcore_barrier()
        pltpu.sync_copy(
            acc.at[pl.ds(s * ZROWS, ZROWS)], out_hbm.at[c, pl.ds(s * ZROWS, ZROWS)]
        )

    return _scat


_scatter128 = _make_scatter(128, jnp.bfloat16, nbuf=5)
_scatter64 = _make_scatter(64, jnp.bfloat16, nbuf=8)


def _tc1_body(x_ref, w_ref, da_ref, db_ref, y16_ref, dinv_ref):
    deg = da_ref[...] + db_ref[...] + 1.0
    dinv = 1.0 / jnp.sqrt(deg)
    y = jnp.dot(x_ref[...], w_ref[...], preferred_element_type=jnp.float32) * dinv
    y16_ref[...] = y.astype(jnp.bfloat16)
    dinv_ref[...] = dinv


def _tc1(x, w1, da, db):
    return pl.pallas_call(
        _tc1_body,
        grid=(GRID,),
        in_specs=[
            pl.BlockSpec((BR, 256), lambda i: (i, 0)),
            pl.BlockSpec((256, 128), lambda i: (0, 0)),
            pl.BlockSpec((BR, 1), lambda i: (i, 0)),
            pl.BlockSpec((BR, 1), lambda i: (i, 0)),
        ],
        out_specs=[
            pl.BlockSpec((BR, 128), lambda i: (i, 0)),
            pl.BlockSpec((BR, 1), lambda i: (i, 0)),
        ],
        out_shape=[
            jax.ShapeDtypeStruct((N_NODES, 128), jnp.bfloat16),
            jax.ShapeDtypeStruct((N_NODES, 1), jnp.float32),
        ],
    )(x, w1, da, db)


def _bn_scale_shift(st_ref, g_ref, bt_ref):
    mean = st_ref[0:1, :] * (1.0 / N_NODES)
    var = st_ref[1:2, :] * (1.0 / N_NODES) - mean * mean
    scale = g_ref[...] / jnp.sqrt(var + EPS)
    shift = bt_ref[...] - mean * scale
    return scale, shift


def _pre_phase(acc_ref, yb_ref, dinv_ref, b_ref, pre_scr, st_scr):
    i = pl.program_id(1)
    a = acc_ref[0].astype(jnp.float32) + acc_ref[1].astype(jnp.float32)
    pre = (a + yb_ref[...].astype(jnp.float32)) * dinv_ref[...] + b_ref[...]
    pre_scr[pl.ds(i * BR, BR), :] = pre
    st = jnp.concatenate(
        [
            jnp.sum(pre, axis=0, keepdims=True),
            jnp.sum(pre * pre, axis=0, keepdims=True),
        ],
        axis=0,
    )

    @pl.when(i == 0)
    def _():
        st_scr[...] = st

    @pl.when(i > 0)
    def _():
        st_scr[...] += st


def _l1_body(acc_ref, yb_ref, dinv_ref, b_ref, g_ref, bt_ref, w_ref,
             h_ref, y2b_ref, pre_scr, st_scr):
    ph = pl.program_id(0)
    i = pl.program_id(1)

    @pl.when(ph == 0)
    def _():
        _pre_phase(acc_ref, yb_ref, dinv_ref, b_ref, pre_scr, st_scr)

    @pl.when(ph == 1)
    def _():
        scale, shift = _bn_scale_shift(st_scr, g_ref, bt_ref)
        h = jnp.maximum(pre_scr[pl.ds(i * BR, BR), :] * scale + shift, 0.0)
        h_ref[...] = h
        y2 = (
            jnp.dot(h, w_ref[...], preferred_element_type=jnp.float32)
            * dinv_ref[...]
        )
        y2b_ref[...] = y2.astype(jnp.bfloat16)


def _l1(acc, y1b, dinv, b, g, bt, w2):
    return pl.pallas_call(
        _l1_body,
        grid=(2, GRID),
        in_specs=[
            pl.BlockSpec((NC, BR, 128), lambda p, i: (0, i, 0)),
            pl.BlockSpec((BR, 128), lambda p, i: (i, 0)),
            pl.BlockSpec((BR, 1), lambda p, i: (i, 0)),
            pl.BlockSpec((1, 128), lambda p, i: (0, 0)),
            pl.BlockSpec((1, 128), lambda p, i: (0, 0)),
            pl.BlockSpec((1, 128), lambda p, i: (0, 0)),
            pl.BlockSpec((128, 64), lambda p, i: (0, 0)),
        ],
        out_specs=[
            pl.BlockSpec((BR, 128), lambda p, i: (i, 0)),
            pl.BlockSpec((BR, 64), lambda p, i: (i, 0)),
        ],
        out_shape=[
            jax.ShapeDtypeStruct((N_NODES, 128), jnp.float32),
            jax.ShapeDtypeStruct((N_NODES, 64), jnp.bfloat16),
        ],
        scratch_shapes=[
            pltpu.VMEM((N_NODES, 128), jnp.float32),
            pltpu.VMEM((2, 128), jnp.float32),
        ],
    )(acc, y1b, dinv, b, g, bt, w2)


def _l2_body(acc_ref, yb_ref, dinv_ref, b_ref, g_ref, bt_ref, x_ref, h1_ref,
             wf_ref, bf_ref, out_ref, pre_scr, st_scr):
    ph = pl.program_id(0)
    i = pl.program_id(1)

    @pl.when(ph == 0)
    def _():
        _pre_phase(acc_ref, yb_ref, dinv_ref, b_ref, pre_scr, st_scr)

    @pl.when(ph == 1)
    def _():
        scale, shift = _bn_scale_shift(st_scr, g_ref, bt_ref)
        h2 = jnp.maximum(pre_scr[pl.ds(i * BR, BR), :] * scale + shift, 0.0)
        o = (
            jnp.dot(x_ref[...], wf_ref[0:256, :], preferred_element_type=jnp.float32)
            + jnp.dot(h1_ref[...], wf_ref[256:384, :], preferred_element_type=jnp.float32)
            + jnp.dot(h2, wf_ref[384:448, :], preferred_element_type=jnp.float32)
            + bf_ref[...]
        )
        out_ref[...] = jnp.maximum(o, 0.0)


def _l2(acc, y2b, dinv, b, g, bt, x, h1, wf, bf):
    return pl.pallas_call(
        _l2_body,
        grid=(2, GRID),
        in_specs=[
            pl.BlockSpec((NC, BR, 64), lambda p, i: (0, i, 0)),
            pl.BlockSpec((BR, 64), lambda p, i: (i, 0)),
            pl.BlockSpec((BR, 1), lambda p, i: (i, 0)),
            pl.BlockSpec((1, 64), lambda p, i: (0, 0)),
            pl.BlockSpec((1, 64), lambda p, i: (0, 0)),
            pl.BlockSpec((1, 64), lambda p, i: (0, 0)),
            pl.BlockSpec((BR, 256), lambda p, i: (i, 0)),
            pl.BlockSpec((BR, 128), lambda p, i: (i, 0)),
            pl.BlockSpec((448, 256), lambda p, i: (0, 0)),
            pl.BlockSpec((1, 256), lambda p, i: (0, 0)),
        ],
        out_specs=pl.BlockSpec((BR, 256), lambda p, i: (i, 0)),
        out_shape=jax.ShapeDtypeStruct((N_NODES, 256), jnp.float32),
        scratch_shapes=[
            pltpu.VMEM((N_NODES, 64), jnp.float32),
            pltpu.VMEM((2, 64), jnp.float32),
        ],
    )(acc, y2b, dinv, b, g, bt, x, h1, wf, bf)


def kernel(node_features, edge_index, W1, b1, g1, bt1, W2, b2, g2, bt2, Wf, bf):
    x = node_features
    ei = edge_index.astype(jnp.int32)
    src3 = jnp.pad(ei[0].reshape(NW, EPW), ((0, 0), (0, EPWP - EPW))).reshape(
        NW, NCHUNK, CH
    )
    dst3 = jnp.pad(
        ei[1].reshape(NW, EPW), ((0, 0), (0, EPWP - EPW)), constant_values=TRASH
    ).reshape(NW, NCHUNK, CH)
    zd = jnp.zeros((ZROWS,), jnp.float32)
    z1 = jnp.zeros((ZROWS, 128), jnp.bfloat16)
    z2 = jnp.zeros((ZROWS, 64), jnp.bfloat16)

    degp = _deg_kernel(dst3, zd)
    da = degp[0, :N_NODES].reshape(N_NODES, 1)
    db = degp[1, :N_NODES].reshape(N_NODES, 1)
    y1b, dinv = _tc1(x, W1, da, db)
    acc1 = _scatter128(y1b, src3, dst3, z1)
    h1, y2b = _l1(
        acc1, y1b, dinv, b1.reshape(1, 128), g1.reshape(1, 128),
        bt1.reshape(1, 128), W2,
    )
    acc2 = _scatter64(y2b, src3, dst3, z2)
    return _l2(
        acc2, y2b, dinv, b2.reshape(1, 64), g2.reshape(1, 64),
        bt2.reshape(1, 64), x, h1, Wf, bf.reshape(1, 256),
    )
